# Initial kernel scaffold; baseline (speedup 1.0000x reference)
import jax
import jax.numpy as jnp
from jax.experimental import pallas as pl


def kernel(x, edge_index, Wl1, bl1, Wr1, g1, be1, Wl2, bl2, Wr2, g2, be2, Wl3, bl3, Wr3):
    raise NotImplementedError("write your pallas kernel here")


# speedup vs baseline: 3.0640x; 3.0640x over previous
import functools
import math

import jax
import jax.numpy as jnp
from jax import lax
from jax.experimental import pallas as pl
from jax.experimental.pallas import tpu as pltpu
from jax.experimental.pallas import tpu_sc as plsc

N = 10000
D = 128
E = 320000
NC = 2
NS = 16
NW = NC * NS
C = 128
CHUNKS = -(-E // (NW * C))
EPW = CHUNKS * C
EPAD = EPW * NW
NPAD = 10240
RPT = NPAD // NS
ZCH = RPT // C

RB = 1000
GRID = N // RB

_BN = float(1.0 / math.sqrt(1.0 + 1e-5))


def _sc_body(p_hbm, src_hbm, dst_hbm, z_hbm,
             agg_hbm, src_v, dst_v, rows_v, agg_sp, sem):
    c = lax.axis_index("c")
    s = lax.axis_index("s")
    wid = c * NS + s
    base = s * RPT

    pltpu.sync_copy(z_hbm, rows_v)
    for j in range(ZCH):
        pltpu.sync_copy(rows_v, agg_sp.at[pl.ds(base + j * C, C)])

    plsc.subcore_barrier()

    def _edge_chunk(j, _):
        pltpu.sync_copy(src_hbm.at[wid, j], src_v)
        pltpu.sync_copy(dst_hbm.at[wid, j], dst_v)
        pltpu.async_copy(p_hbm.at[src_v], rows_v, sem).wait()
        pltpu.sync_copy(rows_v, agg_sp.at[dst_v], add=True)
        return 0
    lax.fori_loop(0, CHUNKS, _edge_chunk, 0)

    plsc.subcore_barrier()

    for j in range(ZCH):
        r0 = base + j * C
        pltpu.sync_copy(agg_sp.at[pl.ds(r0, C)], rows_v)
        pltpu.sync_copy(rows_v, agg_hbm.at[c, pl.ds(r0, C)])


def _make_sc():
    mesh = plsc.VectorSubcoreMesh(core_axis_name="c", subcore_axis_name="s")
    scratch = [
        pltpu.VMEM((C,), jnp.int32),
        pltpu.VMEM((C,), jnp.int32),
        pltpu.VMEM((C, D), jnp.float32),
        pltpu.VMEM_SHARED((NPAD, D), jnp.float32),
        pltpu.SemaphoreType.DMA,
    ]
    return pl.kernel(
        _sc_body,
        out_type=jax.ShapeDtypeStruct((NC, NPAD, D), jnp.float32),
        mesh=mesh,
        scratch_types=scratch,
    )


def _dot_t(a, w):
    return lax.dot_general(a, w, (((1,), (1,)), ((), ())),
                           preferred_element_type=jnp.float32)


def _t1_body(x_ref, wl_ref, wr_ref, bl_ref, p_ref, r_ref):
    x = x_ref[...]
    p_ref[...] = _dot_t(x, wl_ref[...])
    r_ref[...] = _dot_t(x, wr_ref[...]) + bl_ref[...]


def _t2_body(a_ref, dw_ref, rp_ref, g_ref, be_ref, wl_ref, wr_ref, bl_ref,
             p_ref, r_ref):
    a = a_ref[0] + a_ref[1]
    deg = dw_ref[0, :, 0:1] + dw_ref[1, :, 0:1]
    h = a / jnp.maximum(deg, 1.0) + rp_ref[...]
    h = jnp.maximum(h * (g_ref[...] * _BN) + be_ref[...], 0.0)
    p_ref[...] = _dot_t(h, wl_ref[...])
    r_ref[...] = _dot_t(h, wr_ref[...]) + bl_ref[...]


def _t3_body(a_ref, dw_ref, rp_ref, o_ref):
    a = a_ref[0] + a_ref[1]
    deg = dw_ref[0, :, 0:1] + dw_ref[1, :, 0:1]
    o_ref[...] = a / jnp.maximum(deg, 1.0) + rp_ref[...]


_ROWS = pl.BlockSpec((RB, D), lambda i: (i, 0))
_W = pl.BlockSpec((D, D), lambda i: (0, 0))
_VEC = pl.BlockSpec((1, D), lambda i: (0, 0))
_AGG = pl.BlockSpec((NC, RB, D), lambda i: (0, i, 0))
_DEGW = _AGG

_t1 = pl.pallas_call(
    _t1_body,
    grid=(GRID,),
    in_specs=[_ROWS, _W, _W, _VEC],
    out_specs=[_ROWS, _ROWS],
    out_shape=[jax.ShapeDtypeStruct((N, D), jnp.float32)] * 2,
)

_t2 = pl.pallas_call(
    _t2_body,
    grid=(GRID,),
    in_specs=[_AGG, _DEGW, _ROWS, _VEC, _VEC, _W, _W, _VEC],
    out_specs=[_ROWS, _ROWS],
    out_shape=[jax.ShapeDtypeStruct((N, D), jnp.float32)] * 2,
)

_t3 = pl.pallas_call(
    _t3_body,
    grid=(GRID,),
    in_specs=[_AGG, _DEGW, _ROWS],
    out_specs=_ROWS,
    out_shape=jax.ShapeDtypeStruct((N, D), jnp.float32),
)

@functools.lru_cache(maxsize=None)
def _get_sc():
    return _make_sc()


def kernel(x, edge_index, Wl1, bl1, Wr1, g1, be1, Wl2, bl2, Wr2, g2, be2,
           Wl3, bl3, Wr3):
    x2 = x[0]
    pad = EPAD - E
    srcp = jnp.concatenate([edge_index[0], jnp.zeros((pad,), jnp.int32)])
    dstp = jnp.concatenate([edge_index[1], jnp.full((pad,), N, jnp.int32)])
    src3 = srcp.reshape(NW, CHUNKS, C)
    dst3 = dstp.reshape(NW, CHUNKS, C)
    z = jnp.zeros((C, D), jnp.float32)
    ones_tbl = jnp.ones((N, D), jnp.float32)

    bl1r, bl2r, bl3r = bl1[None], bl2[None], bl3[None]
    g1r, be1r = g1[None], be1[None]
    g2r, be2r = g2[None], be2[None]

    sc = _get_sc()
    dw = sc(ones_tbl, src3, dst3, z)
    p1, r1 = _t1(x2, Wl1, Wr1, bl1r)
    a1 = sc(p1, src3, dst3, z)
    p2, r2 = _t2(a1, dw, r1, g1r, be1r, Wl2, Wr2, bl2r)
    a2 = sc(p2, src3, dst3, z)
    p3, r3 = _t2(a2, dw, r2, g2r, be2r, Wl3, Wr3, bl3r)
    a3 = sc(p3, src3, dst3, z)
    out = _t3(a3, dw, r3)
    return out[None]
